# baseline (speedup 1.0000x reference)
import functools

import jax
import jax.numpy as jnp
from jax import lax
from jax.experimental import pallas as pl
from jax.experimental.pallas import tpu as pltpu
from jax.experimental.pallas import tpu_sc as plsc

_VOCAB = 1000000
_HIDDEN = 64
_BATCH = 4096
_SEQ = 200

_TOT = _BATCH * _SEQ
_CH = 128
_NBC = _BATCH // _CH
_NC = 2
_NS = 16
_NW = _NC * _NS
_CPW = _TOT // (_CH * _NW)
_L = 16
_D = 4

_mesh = plsc.VectorSubcoreMesh(core_axis_name="c", subcore_axis_name="s")


@functools.partial(
    pl.kernel,
    out_type=jax.ShapeDtypeStruct((_SEQ, _HIDDEN // 8, _NBC, 8 * _CH), jnp.float32),
    mesh=_mesh,
    compiler_params=pltpu.CompilerParams(
        use_tc_tiling_on_sc=False, needs_layout_passes=False
    ),
    scratch_types=(
        [pltpu.VMEM((_CPW, _CH), jnp.int32)]
        + [pltpu.VMEM((_CH, _HIDDEN), jnp.float32)] * _D
        + [pltpu.VMEM((_CH * _HIDDEN,), jnp.float32)] * _D
        + [pltpu.SemaphoreType.DMA] * _D
        + [pltpu.SemaphoreType.DMA] * _D
    ),
)
def _embed_gather(ids_hbm, table_hbm, out_hbm, idx_v, *bufs):
    buf = bufs[:_D]
    tb = bufs[_D:2 * _D]
    gsem = bufs[2 * _D:3 * _D]
    psem = bufs[3 * _D:4 * _D]

    wid = lax.axis_index("s") * _NC + lax.axis_index("c")
    crow = wid * _CPW
    pltpu.sync_copy(ids_hbm.at[pl.ds(crow, _CPW)], idx_v)

    iota = lax.iota(jnp.int32, _L)
    idx_dst = [(iota + _L * m) * _CH for m in range(_HIDDEN // _L)]

    def gather(c, p):
        return pltpu.make_async_copy(table_hbm.at[idx_v.at[c]], buf[p], gsem[p])

    def drain(p, sem):
        pltpu.make_async_copy(table_hbm.at[pl.ds(0, _CH)], buf[p], sem).wait()

    def transpose(p):
        @plsc.parallel_loop(0, _CH, unroll=16)
        def _(r):
            for m in range(_HIDDEN // _L):
                plsc.store_scatter(tb[p], [idx_dst[m] + r], buf[p][r, pl.ds(_L * m, _L)])

    def put(c, p):
        s = (crow + c) // _NBC
        bc = lax.rem(crow + c, _NBC)
        for jr in range(_HIDDEN // 8):
            pltpu.make_async_copy(
                tb[p].at[pl.ds(jr * 8 * _CH, 8 * _CH)], out_hbm.at[s, jr, bc], psem[p]
            ).start()

    for p in range(_D):
        gather(p, p).start()

    def step(t, carry):
        for p in range(_D):
            c = _D * t + p
            drain(p, gsem[p])

            @pl.when(t > 0)
            def _():
                drain(p, psem[p])

            transpose(p)
            put(c, p)

            @pl.when(t < _CPW // _D - 1)
            def _():
                gather(c + _D, p).start()

        return carry

    lax.fori_loop(0, _CPW // _D, step, 0)
    for p in range(_D):
        drain(p, psem[p])


def kernel(input_ids, token_embeddings):
    ids2 = input_ids.T.reshape(_TOT // _CH, _CH)
    r = _embed_gather(ids2, token_embeddings)
    r = r.reshape(_SEQ, _HIDDEN // 8, _NBC, 8, _CH)
    return r.transpose(2, 4, 0, 1, 3).reshape(_BATCH, _SEQ, _HIDDEN)


# speedup vs baseline: 1.0029x; 1.0029x over previous
import functools

import jax
import jax.numpy as jnp
from jax import lax
from jax.experimental import pallas as pl
from jax.experimental.pallas import tpu as pltpu
from jax.experimental.pallas import tpu_sc as plsc

_VOCAB = 1000000
_HIDDEN = 64
_BATCH = 4096
_SEQ = 200

_TOT = _BATCH * _SEQ
_CH = 128
_NBC = _BATCH // _CH
_NC = 2
_NS = 16
_NW = _NC * _NS
_CPW = _TOT // (_CH * _NW)
_L = 16
_D = 4

_mesh = plsc.VectorSubcoreMesh(core_axis_name="c", subcore_axis_name="s")


@functools.partial(
    pl.kernel,
    out_type=jax.ShapeDtypeStruct((_SEQ, _HIDDEN // 8, _NBC, 8 * _CH), jnp.float32),
    mesh=_mesh,
    compiler_params=pltpu.CompilerParams(
        use_tc_tiling_on_sc=False, needs_layout_passes=False
    ),
    scratch_types=(
        [pltpu.VMEM((_CPW, _CH), jnp.int32)]
        + [pltpu.VMEM((_CH, _HIDDEN), jnp.float32)] * _D
        + [pltpu.VMEM((_CH * _HIDDEN,), jnp.float32)] * _D
        + [pltpu.SemaphoreType.DMA] * _D
        + [pltpu.SemaphoreType.DMA] * _D
    ),
)
def _embed_gather(ids_hbm, table_hbm, out_hbm, idx_v, *bufs):
    buf = bufs[:_D]
    tb = bufs[_D:2 * _D]
    gsem = bufs[2 * _D:3 * _D]
    psem = bufs[3 * _D:4 * _D]

    wid = lax.axis_index("s") * _NC + lax.axis_index("c")
    crow = wid * _CPW
    pltpu.sync_copy(ids_hbm.at[pl.ds(crow, _CPW)], idx_v)

    iota = lax.iota(jnp.int32, _L)
    idx_dst = [(iota + _L * m) * _CH for m in range(_HIDDEN // _L)]

    def gather(c, p):
        return pltpu.make_async_copy(table_hbm.at[idx_v.at[c]], buf[p], gsem[p])

    def drain(p, sem):
        pltpu.make_async_copy(table_hbm.at[pl.ds(0, _CH)], buf[p], sem).wait()

    def transpose(p):
        @plsc.parallel_loop(0, _CH, unroll=8)
        def _(r):
            for m in range(_HIDDEN // _L):
                plsc.store_scatter(tb[p], [idx_dst[m] + r], buf[p][r, pl.ds(_L * m, _L)])

    def put(c, p):
        s = (crow + c) // _NBC
        bc = lax.rem(crow + c, _NBC)
        for jr in range(_HIDDEN // 8):
            pltpu.make_async_copy(
                tb[p].at[pl.ds(jr * 8 * _CH, 8 * _CH)], out_hbm.at[s, jr, bc], psem[p]
            ).start()

    for p in range(_D):
        gather(p, p).start()

    def step(t, carry):
        for p in range(_D):
            c = _D * t + p
            drain(p, gsem[p])

            @pl.when(t > 0)
            def _():
                drain(p, psem[p])

            transpose(p)
            put(c, p)

            @pl.when(t < _CPW // _D - 1)
            def _():
                gather(c + _D, p).start()

        return carry

    lax.fori_loop(0, _CPW // _D, step, 0)
    for p in range(_D):
        drain(p, psem[p])


def kernel(input_ids, token_embeddings):
    ids2 = input_ids.T.reshape(_TOT // _CH, _CH)
    r = _embed_gather(ids2, token_embeddings)
    r = r.reshape(_SEQ, _HIDDEN // 8, _NBC, 8, _CH)
    return r.transpose(2, 4, 0, 1, 3).reshape(_BATCH, _SEQ, _HIDDEN)
